# baseline (speedup 1.0000x reference)
import dataclasses
import functools

import jax
import jax.numpy as jnp
from jax import lax
from jax.experimental import pallas as pl
from jax.experimental.pallas import tpu as pltpu
from jax.experimental.pallas import tpu_sc as plsc

VOCAB = 1_000_000
EMBED_DIM = 64
BATCH = 4096
HIST = 200

_PACK = 4
_K = _PACK * EMBED_DIM
_ROWS = VOCAB // _PACK
_BLK = 2000
_GRID = _ROWS // _BLK


def _proj_body(x_ref, w_ref, o_ref):
    x = x_ref[...].astype(jnp.bfloat16)
    o_ref[...] = lax.dot_general(
        x, w_ref[...],
        dimension_numbers=(((1,), (0,)), ((), ())),
        preferred_element_type=jnp.float32)


def _project_table(emb, w_bd):
    e4 = emb.reshape(_ROWS, _K)
    out = pl.pallas_call(
        _proj_body,
        grid=(_GRID,),
        in_specs=[
            pl.BlockSpec((_BLK, _K), lambda i: (i, 0)),
            pl.BlockSpec((_K, _PACK), lambda i: (0, 0)),
        ],
        out_specs=pl.BlockSpec((_BLK, _PACK), lambda i: (i, 0)),
        out_shape=jax.ShapeDtypeStruct((_ROWS, _PACK), jnp.float32),
    )(e4, w_bd)
    return out.reshape(VOCAB)


_NW = 32
_B_PER_W = BATCH // _NW
_IDX_PER_W = _B_PER_W * HIST
_CHUNK = 128
_NCHUNK = _IDX_PER_W // _CHUNK
_WINDOW = 4


_NSL = _B_PER_W // 16


def _bag_body(proj_hbm, idx_hbm, bias_hbm, out_hbm,
              idx_v, val_v, out_v, bias_v, gsem):
    wid = lax.axis_index("s") * 2 + lax.axis_index("c")
    base = wid * _IDX_PER_W

    pltpu.sync_copy(idx_hbm.at[pl.ds(base, _IDX_PER_W)], idx_v)
    pltpu.sync_copy(bias_hbm, bias_v)

    def _gather(i):
        sl = pl.ds(i * _CHUNK, _CHUNK)
        return pltpu.make_async_copy(proj_hbm.at[idx_v.at[sl]],
                                     val_v.at[sl], gsem)

    for j in range(_WINDOW):
        _gather(j).start()

    @pl.loop(0, _NCHUNK - _WINDOW)
    def _(i):
        _gather(i).wait()
        _gather(i + _WINDOW).start()

    for j in range(_WINDOW):
        _gather(j).wait()

    zero = jnp.zeros((16,), jnp.float32)
    lane = lax.iota(jnp.int32, 16)
    bias = bias_v[...]

    for g in range(_NSL):
        rowbase = (g * 16 + lane) * HIST

        def _acc_body(l, carry, rowbase=rowbase):
            acc, vidx = carry
            return acc + plsc.load_gather(val_v, [vidx]), vidx + 1

        acc, _ = lax.fori_loop(0, HIST, _acc_body, (zero, rowbase))
        x = acc + bias
        out_v[pl.ds(g * 16, 16)] = 1.0 / (1.0 + jnp.exp(-x))

    pltpu.sync_copy(out_v, out_hbm.at[pl.ds(wid * _B_PER_W, _B_PER_W)])


def _bag(proj, idx_flat, bias16):
    mesh = plsc.VectorSubcoreMesh(core_axis_name="c", subcore_axis_name="s")
    cp = pltpu.CompilerParams()
    if "needs_layout_passes" in pltpu.CompilerParams.__dataclass_fields__:
        cp = dataclasses.replace(cp, needs_layout_passes=False)
    kern = pl.kernel(
        out_type=jax.ShapeDtypeStruct((BATCH,), jnp.float32),
        mesh=mesh,
        scratch_types=[
            pltpu.VMEM((_IDX_PER_W,), jnp.int32),
            pltpu.VMEM((_IDX_PER_W,), jnp.float32),
            pltpu.VMEM((_B_PER_W,), jnp.float32),
            pltpu.VMEM((16,), jnp.float32),
            pltpu.SemaphoreType.DMA,
        ],
        compiler_params=cp,
    )(_bag_body)
    return kern(proj, idx_flat, bias16)


def kernel(input_ids, embedding_matrix, linear_w, linear_b):
    w_col = (linear_w.reshape(EMBED_DIM, 1) / HIST).astype(jnp.bfloat16)
    w_bd = jnp.kron(jnp.eye(_PACK, dtype=jnp.bfloat16), w_col)
    proj = _project_table(embedding_matrix, w_bd)
    idx_flat = input_ids.astype(jnp.int32).reshape(BATCH * HIST)
    bias16 = jnp.broadcast_to(linear_b.astype(jnp.float32), (16,))
    out = _bag(proj, idx_flat, bias16)
    return out.reshape(BATCH, 1)


# speedup vs baseline: 5.4447x; 5.4447x over previous
import jax
import jax.numpy as jnp
from jax import lax
from jax.experimental import pallas as pl
from jax.experimental.pallas import tpu as pltpu
from jax.experimental.pallas import tpu_sc as plsc

VOCAB = 1_000_000
EMBED_DIM = 64
BATCH = 4096
HIST = 200

_L = 16384
_TC_GRID = -(-VOCAB // _L)


def _proj_body(x_ref, w_ref, o_ref):
    x = x_ref[...]
    w = w_ref[...]
    o_ref[...] = jnp.sum(x * w, axis=0)


def _project_table(e_t, w_col):
    return pl.pallas_call(
        _proj_body,
        grid=(_TC_GRID,),
        in_specs=[
            pl.BlockSpec((EMBED_DIM, _L), lambda i: (0, i)),
            pl.BlockSpec((EMBED_DIM, 1), lambda i: (0, 0)),
        ],
        out_specs=pl.BlockSpec((_L,), lambda i: (i,)),
        out_shape=jax.ShapeDtypeStruct((VOCAB,), jnp.float32),
    )(e_t, w_col)


_NW = 32
_B_PER_W = BATCH // _NW
_CHUNK = 128
_WINDOW = 4
_NSL = _B_PER_W // 16


def _bag_body(proj_hbm, idst_hbm, bias_hbm, out_hbm,
              idx_v, val_v, out_v, bias_v, gsem):
    wid = lax.axis_index("s") * 2 + lax.axis_index("c")
    base = wid * _B_PER_W

    pltpu.sync_copy(idst_hbm.at[:, pl.ds(base, _B_PER_W)], idx_v)
    pltpu.sync_copy(bias_hbm, bias_v)

    def _gather(l):
        return pltpu.make_async_copy(proj_hbm.at[idx_v.at[l]],
                                     val_v.at[l], gsem)

    for j in range(_WINDOW):
        _gather(j).start()

    @pl.loop(0, HIST - _WINDOW)
    def _(l):
        _gather(l).wait()
        _gather(l + _WINDOW).start()

    for j in range(_WINDOW):
        _gather(j).wait()

    zero = jnp.zeros((16,), jnp.float32)

    def _acc_body(l, accs):
        return tuple(a + val_v[l, pl.ds(s * 16, 16)]
                     for s, a in enumerate(accs))

    accs = lax.fori_loop(0, HIST, _acc_body, (zero,) * _NSL)

    bias = bias_v[...]
    for s in range(_NSL):
        x = accs[s] + bias
        out_v[pl.ds(s * 16, 16)] = 1.0 / (1.0 + jnp.exp(-x))

    pltpu.sync_copy(out_v, out_hbm.at[pl.ds(base, _B_PER_W)])


def _bag(proj, ids_t, bias16):
    mesh = plsc.VectorSubcoreMesh(core_axis_name="c", subcore_axis_name="s")
    kern = pl.kernel(
        out_type=jax.ShapeDtypeStruct((BATCH,), jnp.float32),
        mesh=mesh,
        scratch_types=[
            pltpu.VMEM((HIST, _B_PER_W), jnp.int32),
            pltpu.VMEM((HIST, _B_PER_W), jnp.float32),
            pltpu.VMEM((_B_PER_W,), jnp.float32),
            pltpu.VMEM((16,), jnp.float32),
            pltpu.SemaphoreType.DMA,
        ],
    )(_bag_body)
    return kern(proj, ids_t, bias16)


def kernel(input_ids, embedding_matrix, linear_w, linear_b):
    e_t = embedding_matrix.T
    w_col = linear_w.reshape(EMBED_DIM, 1) / HIST
    proj = _project_table(e_t, w_col)
    ids_t = input_ids.astype(jnp.int32).T
    bias16 = jnp.broadcast_to(linear_b.astype(jnp.float32), (16,))
    out = _bag(proj, ids_t, bias16)
    return out.reshape(BATCH, 1)


# speedup vs baseline: 6.1444x; 1.1285x over previous
import jax
import jax.numpy as jnp
from jax import lax
from jax.experimental import pallas as pl
from jax.experimental.pallas import tpu as pltpu
from jax.experimental.pallas import tpu_sc as plsc

VOCAB = 1_000_000
EMBED_DIM = 64
BATCH = 4096
HIST = 200

_L = 16384
_TC_GRID = -(-VOCAB // _L)


def _proj_body(x_ref, w_ref, o_ref):
    x = x_ref[...]
    w = w_ref[...]
    o_ref[...] = jnp.sum(x * w, axis=0)


def _project_table(e_t, w_col):
    return pl.pallas_call(
        _proj_body,
        grid=(_TC_GRID,),
        in_specs=[
            pl.BlockSpec((EMBED_DIM, _L), lambda i: (0, i)),
            pl.BlockSpec((EMBED_DIM, 1), lambda i: (0, 0)),
        ],
        out_specs=pl.BlockSpec((_L,), lambda i: (i,)),
        out_shape=jax.ShapeDtypeStruct((VOCAB,), jnp.float32),
    )(e_t, w_col)


_NW = 32
_B_PER_W = BATCH // _NW
_CHUNK = 128
_WINDOW = 16
_NSL = _B_PER_W // 16


def _bag_body(proj_hbm, idst_hbm, bias_hbm, out_hbm,
              idx_v, val_v, out_v, bias_v, gsem):
    wid = lax.axis_index("s") * 2 + lax.axis_index("c")
    base = wid * _B_PER_W

    pltpu.sync_copy(idst_hbm.at[:, pl.ds(base, _B_PER_W)], idx_v)
    pltpu.sync_copy(bias_hbm, bias_v)

    def _gather(l):
        return pltpu.make_async_copy(proj_hbm.at[idx_v.at[l]],
                                     val_v.at[l], gsem)

    for j in range(_WINDOW):
        _gather(j).start()

    @pl.loop(0, HIST - _WINDOW)
    def _(l):
        _gather(l).wait()
        _gather(l + _WINDOW).start()

    for j in range(_WINDOW):
        _gather(j).wait()

    zero = jnp.zeros((16,), jnp.float32)

    def _acc_body(l, accs):
        return tuple(a + val_v[l, pl.ds(s * 16, 16)]
                     for s, a in enumerate(accs))

    accs = lax.fori_loop(0, HIST, _acc_body, (zero,) * _NSL)

    bias = bias_v[...]
    for s in range(_NSL):
        x = accs[s] + bias
        out_v[pl.ds(s * 16, 16)] = 1.0 / (1.0 + jnp.exp(-x))

    pltpu.sync_copy(out_v, out_hbm.at[pl.ds(base, _B_PER_W)])


def _bag(proj, ids_t, bias16):
    mesh = plsc.VectorSubcoreMesh(core_axis_name="c", subcore_axis_name="s")
    kern = pl.kernel(
        out_type=jax.ShapeDtypeStruct((BATCH,), jnp.float32),
        mesh=mesh,
        scratch_types=[
            pltpu.VMEM((HIST, _B_PER_W), jnp.int32),
            pltpu.VMEM((HIST, _B_PER_W), jnp.float32),
            pltpu.VMEM((_B_PER_W,), jnp.float32),
            pltpu.VMEM((16,), jnp.float32),
            pltpu.SemaphoreType.DMA,
        ],
    )(_bag_body)
    return kern(proj, ids_t, bias16)


def kernel(input_ids, embedding_matrix, linear_w, linear_b):
    e_t = embedding_matrix.T
    w_col = linear_w.reshape(EMBED_DIM, 1) / HIST
    proj = _project_table(e_t, w_col)
    ids_t = input_ids.astype(jnp.int32).T
    bias16 = jnp.broadcast_to(linear_b.astype(jnp.float32), (16,))
    out = _bag(proj, ids_t, bias16)
    return out.reshape(BATCH, 1)


# speedup vs baseline: 6.9554x; 1.1320x over previous
import jax
import jax.numpy as jnp
from jax import lax
from jax.experimental import pallas as pl
from jax.experimental.pallas import tpu as pltpu
from jax.experimental.pallas import tpu_sc as plsc

VOCAB = 1_000_000
EMBED_DIM = 64
BATCH = 4096
HIST = 200

_L = 32768
_TC_GRID = -(-VOCAB // _L)


def _proj_body(x_ref, w_ref, o_ref):
    x = x_ref[...]
    w = w_ref[...]
    o_ref[...] = jnp.sum(x * w, axis=0)


def _project_table(e_t, w_col):
    return pl.pallas_call(
        _proj_body,
        grid=(_TC_GRID,),
        in_specs=[
            pl.BlockSpec((EMBED_DIM, _L), lambda i: (0, i)),
            pl.BlockSpec((EMBED_DIM, 1), lambda i: (0, 0)),
        ],
        out_specs=pl.BlockSpec((_L,), lambda i: (i,)),
        out_shape=jax.ShapeDtypeStruct((VOCAB,), jnp.float32),
    )(e_t, w_col)


_NW = 32
_B_PER_W = BATCH // _NW
_CHUNK = 128
_WINDOW = 32
_NSL = _B_PER_W // 16


def _bag_body(proj_hbm, idst_hbm, bias_hbm, out_hbm,
              idx_v, val_v, out_v, bias_v, gsem):
    wid = lax.axis_index("s") * 2 + lax.axis_index("c")
    base = wid * _B_PER_W

    pltpu.sync_copy(idst_hbm.at[:, pl.ds(base, _B_PER_W)], idx_v)
    pltpu.sync_copy(bias_hbm, bias_v)

    def _gather(l):
        return pltpu.make_async_copy(proj_hbm.at[idx_v.at[l]],
                                     val_v.at[l], gsem)

    for j in range(_WINDOW):
        _gather(j).start()

    @pl.loop(0, HIST - _WINDOW)
    def _(l):
        _gather(l).wait()
        _gather(l + _WINDOW).start()

    for j in range(_WINDOW):
        _gather(j).wait()

    zero = jnp.zeros((16,), jnp.float32)

    def _acc_body(l, accs):
        return tuple(a + val_v[l, pl.ds(s * 16, 16)]
                     for s, a in enumerate(accs))

    accs = lax.fori_loop(0, HIST, _acc_body, (zero,) * _NSL)

    bias = bias_v[...]
    for s in range(_NSL):
        x = accs[s] + bias
        out_v[pl.ds(s * 16, 16)] = 1.0 / (1.0 + jnp.exp(-x))

    pltpu.sync_copy(out_v, out_hbm.at[pl.ds(base, _B_PER_W)])


def _bag(proj, ids_t, bias16):
    mesh = plsc.VectorSubcoreMesh(core_axis_name="c", subcore_axis_name="s")
    kern = pl.kernel(
        out_type=jax.ShapeDtypeStruct((BATCH,), jnp.float32),
        mesh=mesh,
        scratch_types=[
            pltpu.VMEM((HIST, _B_PER_W), jnp.int32),
            pltpu.VMEM((HIST, _B_PER_W), jnp.float32),
            pltpu.VMEM((_B_PER_W,), jnp.float32),
            pltpu.VMEM((16,), jnp.float32),
            pltpu.SemaphoreType.DMA,
        ],
    )(_bag_body)
    return kern(proj, ids_t, bias16)


def kernel(input_ids, embedding_matrix, linear_w, linear_b):
    e_t = embedding_matrix.T
    w_col = linear_w.reshape(EMBED_DIM, 1) / HIST
    proj = _project_table(e_t, w_col)
    ids_t = input_ids.astype(jnp.int32).T
    bias16 = jnp.broadcast_to(linear_b.astype(jnp.float32), (16,))
    out = _bag(proj, ids_t, bias16)
    return out.reshape(BATCH, 1)
